# baseline (speedup 1.0000x reference)
import jax
import jax.numpy as jnp
from jax import lax
from jax.experimental import pallas as pl
from jax.experimental.pallas import tpu as pltpu

_C = 80
_SCALES = ((52, 52), (26, 26), (13, 13))
_B = 32
_G = 20
_A = 3
_BB = 4
_NS = _B // _BB


def _softplus(x):
    return jnp.maximum(x, 0.0) + jnp.log1p(jnp.exp(-jnp.abs(x)))


def _loss_kernel(s3, oa3, bb3, s4, oa4, bb4, s5, oa5, bb5,
                 cr3, cr4, cr5, lr, bxt,
                 out_ref, acc_ref):
    b = pl.program_id(0)

    @pl.when(b == 0)
    def _init():
        for i in range(12):
            acc_ref[i] = 0.0

    gidx_r = lax.broadcasted_iota(jnp.int32, (_G, _G), 0)
    gidx_c = lax.broadcasted_iota(jnp.int32, (_G, _G), 1)
    later_r = gidx_r > gidx_c
    cls_iota = lax.broadcasted_iota(jnp.int32, (_C, _G), 0)

    for i in range(_BB):
        lrow = lr[i]
        boxes_t = bxt[i]
        oh_lab = (cls_iota == lrow).astype(jnp.float32)
        same_lab = lrow.reshape(_G, 1) == lrow

        for s, (slab_ref, obj1_ref, bbox_ref, crow_ref) in enumerate((
                (s3, oa3, bb3, cr3), (s4, oa4, bb4, cr4),
                (s5, oa5, bb5, cr5))):
            H, W = _SCALES[s]
            HW = H * W

            slab = slab_ref[i]
            acc_ref[s] = (acc_ref[s]
                          + jnp.sum(_softplus(
                              slab[0:1, :].astype(jnp.float32)))
                          + jnp.sum(_softplus(obj1_ref[i])))

            crow = crow_ref[i]
            sel = (lax.broadcasted_iota(jnp.int32, (HW, _G), 0)
                   == crow).astype(jnp.float32)
            cv = lax.dot_general(slab, sel.astype(jnp.bfloat16),
                                 (((1,), (0,)), ((), ())),
                                 preferred_element_type=jnp.float32)
            vb = lax.dot_general(bbox_ref[i], sel,
                                 (((1,), (0,)), ((), ())),
                                 preferred_element_type=jnp.float32)
            v0 = cv[0:1, :]
            vc = cv[5:85, :]

            same_cell = crow.reshape(_G, 1) == crow
            winner = 1.0 - jnp.max(
                (same_cell & later_r).astype(jnp.float32),
                axis=0, keepdims=True)
            pairw = 1.0 - jnp.max((same_cell & same_lab & later_r)
                                  .astype(jnp.float32),
                                  axis=0, keepdims=True)
            cnt = jnp.maximum(jnp.sum(winner), 1.0)

            acc_ref[3 + s] = acc_ref[3 + s] + jnp.sum(winner * v0)

            gxf = (crow % W).astype(jnp.float32)
            gyf = (crow // W).astype(jnp.float32)
            tx = boxes_t[0:1, :] * W - gxf
            ty = boxes_t[1:2, :] * H - gyf
            tv = jnp.concatenate(
                [tx, ty, boxes_t[2:3, :], boxes_t[3:4, :]],
                axis=0)
            mse = jnp.sum((vb - tv) ** 2, axis=0, keepdims=True)
            acc_ref[6 + s] = (acc_ref[6 + s]
                              + jnp.sum(winner * mse) / (cnt * 4.0))

            spsum = jnp.sum(_softplus(vc), axis=0, keepdims=True)
            xlab = jnp.sum(vc * oh_lab, axis=0, keepdims=True)
            acc_ref[9 + s] = acc_ref[9 + s] + (
                jnp.sum(winner * spsum) - jnp.sum(pairw * xlab)) / (cnt * _C)

    @pl.when(b == pl.num_programs(0) - 1)
    def _fin():
        to = 0.0
        for s, (H, W) in enumerate(_SCALES):
            to = to + (acc_ref[s] - acc_ref[3 + s]) / (_B * _A * H * W)
        to = to / 3.0
        tb = (acc_ref[6] + acc_ref[7] + acc_ref[8]) / (_B * _G * 3.0)
        tc = (acc_ref[9] + acc_ref[10] + acc_ref[11]) / (_B * _G * 3.0)
        out_ref[0] = to + 5.0 * tb + 2.0 * tc
        out_ref[1] = to
        out_ref[2] = tb
        out_ref[3] = tc


def kernel(p3, p4, p5, targets_boxes, targets_labels):
    preds, in_specs, crows = [], [], []
    for pred, (H, W) in zip((p3, p4, p5), _SCALES):
        slab = pred[:, :85].astype(jnp.bfloat16).reshape(_B, 85, H * W)
        objx = jnp.concatenate(
            [pred[:, 85:86], pred[:, 170:171]], axis=1).reshape(_B, 2, H * W)
        bbx = pred[:, 1:5].reshape(_B, 4, H * W)
        preds.extend([slab, objx, bbx])
        in_specs.append(pl.BlockSpec((_BB, 85, H * W), lambda b: (b, 0, 0)))
        in_specs.append(pl.BlockSpec((_BB, 2, H * W), lambda b: (b, 0, 0)))
        in_specs.append(pl.BlockSpec((_BB, 4, H * W), lambda b: (b, 0, 0)))
        cx = targets_boxes[..., 0]
        cy = targets_boxes[..., 1]
        gx = jnp.clip((cx * W).astype(jnp.int32), 0, W - 1)
        gy = jnp.clip((cy * H).astype(jnp.int32), 0, H - 1)
        crows.append((gy * W + gx)[:, None, :])
    labs = targets_labels.astype(jnp.int32)[:, None, :]
    boxes_t = jnp.transpose(targets_boxes, (0, 2, 1))

    for _ in range(3):
        in_specs.append(pl.BlockSpec((_BB, 1, _G), lambda b: (b, 0, 0)))
    in_specs.append(pl.BlockSpec((_BB, 1, _G), lambda b: (b, 0, 0)))
    in_specs.append(pl.BlockSpec((_BB, 4, _G), lambda b: (b, 0, 0)))

    out = pl.pallas_call(
        _loss_kernel,
        grid=(_NS,),
        in_specs=in_specs,
        out_specs=pl.BlockSpec(memory_space=pltpu.SMEM),
        out_shape=jax.ShapeDtypeStruct((4,), jnp.float32),
        scratch_shapes=[pltpu.SMEM((12,), jnp.float32)],
        compiler_params=pltpu.CompilerParams(
            dimension_semantics=("arbitrary",)),
    )(*preds, *crows, labs, boxes_t)
    return (out[0], out[1], out[2], out[3])


# speedup vs baseline: 1.6760x; 1.6760x over previous
import jax
import jax.numpy as jnp
from jax import lax
from jax.experimental import pallas as pl
from jax.experimental.pallas import tpu as pltpu

_C = 80
_SCALES = ((52, 52), (26, 26), (13, 13))
_B = 32
_G = 20
_A = 3
_BB = 4
_NS = _B // _BB


def _softplus(x):
    return jnp.maximum(x, 0.0) + jnp.log1p(jnp.exp(-jnp.abs(x)))


def _loss_kernel(s3, oa3, s4, oa4, s5, oa5,
                 cr3, cr4, cr5, lr, bxt,
                 out_ref, acc_ref):
    b = pl.program_id(0)

    @pl.when(b == 0)
    def _init():
        for i in range(12):
            acc_ref[i] = 0.0

    gidx_r = lax.broadcasted_iota(jnp.int32, (_G, _G), 0)
    gidx_c = lax.broadcasted_iota(jnp.int32, (_G, _G), 1)
    later_r = gidx_r > gidx_c
    cls_iota = lax.broadcasted_iota(jnp.int32, (_C, _G), 0)

    for i in range(_BB):
        lrow = lr[i]
        boxes_t = bxt[i]
        oh_lab = (cls_iota == lrow).astype(jnp.float32)
        same_lab = lrow.reshape(_G, 1) == lrow

        for s, (slab_ref, obj1_ref, crow_ref) in enumerate((
                (s3, oa3, cr3), (s4, oa4, cr4), (s5, oa5, cr5))):
            H, W = _SCALES[s]
            HW = H * W

            slab = slab_ref[i]
            acc_ref[s] = (acc_ref[s] + jnp.sum(_softplus(slab[0:1, :]))
                          + jnp.sum(_softplus(slab[85:86, :]))
                          + jnp.sum(_softplus(obj1_ref[i])))

            crow = crow_ref[i]
            sel = (lax.broadcasted_iota(jnp.int32, (HW, _G), 0)
                   == crow).astype(jnp.float32)
            cv = lax.dot_general(slab, sel, (((1,), (0,)), ((), ())),
                                 preferred_element_type=jnp.float32)
            v0 = cv[0:1, :]
            vb = cv[1:5, :]
            vc = cv[5:85, :]

            same_cell = crow.reshape(_G, 1) == crow
            winner = 1.0 - jnp.max(
                (same_cell & later_r).astype(jnp.float32),
                axis=0, keepdims=True)
            pairw = 1.0 - jnp.max((same_cell & same_lab & later_r)
                                  .astype(jnp.float32),
                                  axis=0, keepdims=True)
            cnt = jnp.maximum(jnp.sum(winner), 1.0)

            acc_ref[3 + s] = acc_ref[3 + s] + jnp.sum(winner * v0)

            gxf = (crow % W).astype(jnp.float32)
            gyf = (crow // W).astype(jnp.float32)
            tx = boxes_t[0:1, :] * W - gxf
            ty = boxes_t[1:2, :] * H - gyf
            tv = jnp.concatenate(
                [tx, ty, boxes_t[2:3, :], boxes_t[3:4, :]],
                axis=0)
            mse = jnp.sum((vb - tv) ** 2, axis=0, keepdims=True)
            acc_ref[6 + s] = (acc_ref[6 + s]
                              + jnp.sum(winner * mse) / (cnt * 4.0))

            spsum = jnp.sum(_softplus(vc), axis=0, keepdims=True)
            xlab = jnp.sum(vc * oh_lab, axis=0, keepdims=True)
            acc_ref[9 + s] = acc_ref[9 + s] + (
                jnp.sum(winner * spsum) - jnp.sum(pairw * xlab)) / (cnt * _C)

    @pl.when(b == pl.num_programs(0) - 1)
    def _fin():
        to = 0.0
        for s, (H, W) in enumerate(_SCALES):
            to = to + (acc_ref[s] - acc_ref[3 + s]) / (_B * _A * H * W)
        to = to / 3.0
        tb = (acc_ref[6] + acc_ref[7] + acc_ref[8]) / (_B * _G * 3.0)
        tc = (acc_ref[9] + acc_ref[10] + acc_ref[11]) / (_B * _G * 3.0)
        out_ref[0] = to + 5.0 * tb + 2.0 * tc
        out_ref[1] = to
        out_ref[2] = tb
        out_ref[3] = tc


def kernel(p3, p4, p5, targets_boxes, targets_labels):
    preds, in_specs, crows = [], [], []
    for pred, (H, W) in zip((p3, p4, p5), _SCALES):
        slab = pred[:, :88].reshape(_B, 88, H * W)
        objx = pred[:, 170:171].reshape(_B, 1, H * W)
        preds.extend([slab, objx])
        in_specs.append(pl.BlockSpec((_BB, 88, H * W), lambda b: (b, 0, 0)))
        in_specs.append(pl.BlockSpec((_BB, 1, H * W), lambda b: (b, 0, 0)))
        cx = targets_boxes[..., 0]
        cy = targets_boxes[..., 1]
        gx = jnp.clip((cx * W).astype(jnp.int32), 0, W - 1)
        gy = jnp.clip((cy * H).astype(jnp.int32), 0, H - 1)
        crows.append((gy * W + gx)[:, None, :])
    labs = targets_labels.astype(jnp.int32)[:, None, :]
    boxes_t = jnp.transpose(targets_boxes, (0, 2, 1))

    for _ in range(3):
        in_specs.append(pl.BlockSpec((_BB, 1, _G), lambda b: (b, 0, 0)))
    in_specs.append(pl.BlockSpec((_BB, 1, _G), lambda b: (b, 0, 0)))
    in_specs.append(pl.BlockSpec((_BB, 4, _G), lambda b: (b, 0, 0)))

    out = pl.pallas_call(
        _loss_kernel,
        grid=(_NS,),
        in_specs=in_specs,
        out_specs=pl.BlockSpec(memory_space=pltpu.SMEM),
        out_shape=jax.ShapeDtypeStruct((4,), jnp.float32),
        scratch_shapes=[pltpu.SMEM((12,), jnp.float32)],
        compiler_params=pltpu.CompilerParams(
            dimension_semantics=("arbitrary",)),
    )(*preds, *crows, labs, boxes_t)
    return (out[0], out[1], out[2], out[3])
